# baseline (speedup 1.0000x reference)
import functools

import jax
import jax.numpy as jnp
import numpy as np
from jax import lax
from jax.experimental import pallas as pl
from jax.experimental.pallas import tpu as pltpu
from jax.experimental.pallas import tpu_sc as plsc

_PROB = 0.1


@functools.cache
def _make_format_kernel(V, D):
    info = plsc.get_sparse_core_info()
    NC, NS, L = info.num_cores, info.num_subcores, info.num_lanes
    NW = NC * NS
    assert D == 64 and L == 16
    NT = V // 128
    ntail = V - NT * 128
    base_n = NT // NW
    extra = NT - base_n * NW
    mesh = plsc.VectorSubcoreMesh(core_axis_name="c", subcore_axis_name="s")

    @functools.partial(
        pl.kernel,
        mesh=mesh,
        out_type=jax.ShapeDtypeStruct((V // 2 + ntail // 2, 2 * D),
                                      jnp.float32),
        scratch_types=[
            pltpu.VMEM((D, 128), jnp.float32),
            pltpu.VMEM((D, 128), jnp.float32),
            pltpu.VMEM((D, 128), jnp.float32),
            pltpu.VMEM((D, 128), jnp.float32),
            pltpu.VMEM((128,), jnp.float32),
            pltpu.VMEM((128,), jnp.float32),
            pltpu.SemaphoreType.DMA,
            pltpu.SemaphoreType.DMA,
        ],
        compiler_params=pltpu.CompilerParams(
            use_tc_tiling_on_sc=True, needs_layout_passes=False),
    )
    def k(wt_hbm, keep_hbm, tail_hbm, w2_hbm,
          in0, in1, ob0, ob1, kv0, kv1, sem0, sem1):
        wid = lax.axis_index("s") * NC + lax.axis_index("c")
        n_j = base_n + jnp.where(wid < extra, 1, 0)
        ins = (in0, in1)
        obs = (ob0, ob1)
        kvs = (kv0, kv1)
        sems = (sem0, sem1)

        iota = lax.iota(jnp.int32, L)

        def vt_of(j):
            return j * NW + wid

        def fire(j, b):
            c = vt_of(j) * 128
            pltpu.async_copy(wt_hbm.at[:, pl.ds(c, 128)], ins[b], sems[b])
            pltpu.sync_copy(keep_hbm.at[pl.ds(c, 128)], kvs[b])

        def transpose_scale(src, kv, dst, width):
            for g in range(width // L):
                row16 = iota + g * L
                wr16 = jnp.right_shift(row16, 1)
                par64 = jnp.bitwise_and(row16, 1) * D
                k16 = kv[pl.ds(g * L, L)]

                def dbody(dd, carry):
                    cnt = jnp.bitwise_xor(iota, dd)
                    r16 = plsc.load_gather(src, [cnt, row16])
                    plsc.store_scatter(dst, [wr16, par64 + cnt], r16 * k16)
                    return carry

                lax.fori_loop(0, D, dbody, 0, unroll=8)

        def body(j, b):
            @pl.when(j + 1 < n_j)
            def _():
                fire(j + 1, 1 - b)

            pltpu.make_async_copy(
                wt_hbm.at[:, pl.ds(0, 128)], ins[b], sems[b]).wait()
            transpose_scale(ins[b], kvs[b], obs[b], 128)
            pltpu.sync_copy(obs[b], w2_hbm.at[pl.ds(vt_of(j) * 64, 64)])

        fire(0, 0)

        def pair(t, carry):
            body(2 * t, 0)
            body(2 * t + 1, 1)
            return carry

        assert base_n % 2 == 0
        lax.fori_loop(0, base_n // 2, pair, 0)

        @pl.when(wid < extra)
        def _():
            b = base_n % 2
            pltpu.make_async_copy(
                wt_hbm.at[:, pl.ds(0, 128)], ins[b], sems[b]).wait()
            transpose_scale(ins[b], kvs[b], obs[b], 128)
            pltpu.sync_copy(obs[b], w2_hbm.at[pl.ds(vt_of(base_n) * 64, 64)])

        @pl.when(wid == 0)
        def _():
            pltpu.sync_copy(tail_hbm, ob0.at[pl.ds(0, ntail // 2)])
            pltpu.sync_copy(ob0.at[pl.ds(0, ntail // 2)],
                            w2_hbm.at[pl.ds(NT * 64, ntail // 2)])

    return k


@functools.cache
def _make_gather_kernel(V, D, B, H):
    info = plsc.get_sparse_core_info()
    NC, NS, L = info.num_cores, info.num_subcores, info.num_lanes
    NW = NC * NS
    assert D == 64 and L == 16 and B % (128 * NW) == 0 and H % 2 == 0
    n_b = B // NW
    n_i = n_b * H
    mesh = plsc.VectorSubcoreMesh(core_axis_name="c", subcore_axis_name="s")

    @functools.partial(
        pl.kernel,
        mesh=mesh,
        out_type=jax.ShapeDtypeStruct((H, D, B), jnp.float32),
        scratch_types=[
            pltpu.VMEM((n_i,), jnp.int32),
            pltpu.VMEM((n_b, 2 * D), jnp.float32),
            pltpu.VMEM((n_b, 2 * D), jnp.float32),
            pltpu.VMEM((D, n_b), jnp.float32),
            pltpu.VMEM((D, n_b), jnp.float32),
            pltpu.VMEM((n_b,), jnp.int32),
            pltpu.VMEM((n_b,), jnp.int32),
            pltpu.VMEM((n_b,), jnp.int32),
            pltpu.VMEM((n_b,), jnp.int32),
            pltpu.SemaphoreType.DMA,
            pltpu.SemaphoreType.DMA,
        ],
        compiler_params=pltpu.CompilerParams(
            use_tc_tiling_on_sc=True, needs_layout_passes=False),
    )
    def k(w2_hbm, idx_hbm, out_hbm,
          idx_all, rows0, rows1, ot0, ot1, iw0, iw1, po0, po1, sem0, sem1):
        wid = lax.axis_index("s") * NC + lax.axis_index("c")
        pltpu.sync_copy(idx_hbm.at[pl.ds(wid * n_i, n_i)], idx_all)
        rows = (rows0, rows1)
        ots = (ot0, ot1)
        iws = (iw0, iw1)
        pos = (po0, po1)
        sems = (sem0, sem1)

        iota = lax.iota(jnp.int32, L)
        iotaH = iota * H

        def build(h, b):
            for g in range(n_b // L):
                voff = iotaH + (g * L * H + h)
                vo = plsc.load_gather(idx_all, [voff])
                iws[b][pl.ds(g * L, L)] = jnp.right_shift(vo, 1)
                pos[b][pl.ds(g * L, L)] = jnp.bitwise_and(vo, 1) * D

        def fire(b):
            pltpu.async_copy(w2_hbm.at[iws[b]], rows[b], sems[b])

        def compute(h, b):
            rv, ov = rows[b], ots[b]
            for g in range(n_b // L):
                row16 = iota + g * L
                p16 = pos[b][pl.ds(g * L, L)]

                def dbody(dd, carry):
                    cnt = jnp.bitwise_xor(iota, dd)
                    r16 = plsc.load_gather(rv, [row16, p16 + cnt])
                    plsc.store_scatter(ov, [cnt, row16], r16)
                    return carry

                lax.fori_loop(0, D, dbody, 0, unroll=8)
            pltpu.sync_copy(ov, out_hbm.at[h, :, pl.ds(wid * n_b, n_b)])

        build(0, 0)
        fire(0)

        def half(h, b):
            @pl.when(h + 1 < H)
            def _():
                build(h + 1, 1 - b)
                fire(1 - b)

            pltpu.make_async_copy(
                w2_hbm.at[pl.ds(0, n_b)], rows[b], sems[b]).wait()
            compute(h, b)

        def pair(t, carry):
            half(2 * t, 0)
            half(2 * t + 1, 1)
            return carry

        lax.fori_loop(0, H // 2, pair, 0)

    return k


def kernel(x, W):
    B, H = x.shape
    V, D = W.shape
    keep = jax.random.bernoulli(
        jax.random.key(42), 1.0 - _PROB, (V, 1)
    ).astype(W.dtype) / (1.0 - _PROB)
    keep = keep.reshape(V)
    idx = x.reshape(B * H).astype(jnp.int32)
    ntail = V - V // 128 * 128
    tail = (W[V - ntail:] * keep[V - ntail:, None]).reshape(ntail // 2, 2 * D)
    w2 = _make_format_kernel(V, D)(W.T, keep, tail)
    out3 = _make_gather_kernel(V, D, B, H)(w2, idx)
    return jnp.transpose(out3, (2, 0, 1))


# speedup vs baseline: 1.1030x; 1.1030x over previous
import functools

import jax
import jax.numpy as jnp
import numpy as np
from jax import lax
from jax.experimental import pallas as pl
from jax.experimental.pallas import tpu as pltpu
from jax.experimental.pallas import tpu_sc as plsc

_PROB = 0.1


@functools.cache
def _make_format_kernel(V, D):
    info = plsc.get_sparse_core_info()
    NC, NS, L = info.num_cores, info.num_subcores, info.num_lanes
    NW = NC * NS
    assert D == 64 and L == 16
    NT = V // 128
    ntail = V - NT * 128
    base_n = NT // NW
    extra = NT - base_n * NW
    mesh = plsc.VectorSubcoreMesh(core_axis_name="c", subcore_axis_name="s")

    @functools.partial(
        pl.kernel,
        mesh=mesh,
        out_type=jax.ShapeDtypeStruct((V // 2 + ntail // 2, 2 * D),
                                      jnp.float32),
        scratch_types=[
            pltpu.VMEM((D, 128), jnp.float32),
            pltpu.VMEM((D, 128), jnp.float32),
            pltpu.VMEM((D, 128), jnp.float32),
            pltpu.VMEM((D, 128), jnp.float32),
            pltpu.VMEM((128,), jnp.float32),
            pltpu.VMEM((128,), jnp.float32),
            pltpu.SemaphoreType.DMA,
            pltpu.SemaphoreType.DMA,
        ],
        compiler_params=pltpu.CompilerParams(
            use_tc_tiling_on_sc=True, needs_layout_passes=False),
    )
    def k(wt_hbm, keep_hbm, tail_hbm, w2_hbm,
          in0, in1, ob0, ob1, kv0, kv1, sem0, sem1):
        wid = lax.axis_index("s") * NC + lax.axis_index("c")
        n_j = base_n + jnp.where(wid < extra, 1, 0)
        ins = (in0, in1)
        obs = (ob0, ob1)
        kvs = (kv0, kv1)
        sems = (sem0, sem1)

        iota = lax.iota(jnp.int32, L)

        def vt_of(j):
            return j * NW + wid

        def fire(j, b):
            c = vt_of(j) * 128
            pltpu.async_copy(wt_hbm.at[:, pl.ds(c, 128)], ins[b], sems[b])
            pltpu.sync_copy(keep_hbm.at[pl.ds(c, 128)], kvs[b])

        def transpose_scale(src, kv, dst, width):
            ng = width // L
            k16s = [kv[pl.ds(g * L, L)] for g in range(ng)]

            def dbody(dd, carry):
                cnt = jnp.bitwise_xor(iota, dd)
                for g in range(ng):
                    row16 = iota + g * L
                    wr16 = jnp.right_shift(row16, 1)
                    par64 = jnp.bitwise_and(row16, 1) * D
                    r16 = plsc.load_gather(src, [cnt, row16])
                    plsc.store_scatter(dst, [wr16, par64 + cnt],
                                       r16 * k16s[g])
                return carry

            lax.fori_loop(0, D, dbody, 0, unroll=2)

        def body(j, b):
            @pl.when(j + 1 < n_j)
            def _():
                fire(j + 1, 1 - b)

            pltpu.make_async_copy(
                wt_hbm.at[:, pl.ds(0, 128)], ins[b], sems[b]).wait()
            transpose_scale(ins[b], kvs[b], obs[b], 128)
            pltpu.sync_copy(obs[b], w2_hbm.at[pl.ds(vt_of(j) * 64, 64)])

        fire(0, 0)

        def pair(t, carry):
            body(2 * t, 0)
            body(2 * t + 1, 1)
            return carry

        assert base_n % 2 == 0
        lax.fori_loop(0, base_n // 2, pair, 0)

        @pl.when(wid < extra)
        def _():
            b = base_n % 2
            pltpu.make_async_copy(
                wt_hbm.at[:, pl.ds(0, 128)], ins[b], sems[b]).wait()
            transpose_scale(ins[b], kvs[b], obs[b], 128)
            pltpu.sync_copy(obs[b], w2_hbm.at[pl.ds(vt_of(base_n) * 64, 64)])

        @pl.when(wid == 0)
        def _():
            pltpu.sync_copy(tail_hbm, ob0.at[pl.ds(0, ntail // 2)])
            pltpu.sync_copy(ob0.at[pl.ds(0, ntail // 2)],
                            w2_hbm.at[pl.ds(NT * 64, ntail // 2)])

    return k


@functools.cache
def _make_gather_kernel(V, D, B, H):
    info = plsc.get_sparse_core_info()
    NC, NS, L = info.num_cores, info.num_subcores, info.num_lanes
    NW = NC * NS
    assert D == 64 and L == 16 and B % (128 * NW) == 0 and H % 2 == 0
    n_b = B // NW
    n_i = n_b * H
    mesh = plsc.VectorSubcoreMesh(core_axis_name="c", subcore_axis_name="s")

    @functools.partial(
        pl.kernel,
        mesh=mesh,
        out_type=jax.ShapeDtypeStruct((H, D, B), jnp.float32),
        scratch_types=[
            pltpu.VMEM((n_i,), jnp.int32),
            pltpu.VMEM((n_b, 2 * D), jnp.float32),
            pltpu.VMEM((n_b, 2 * D), jnp.float32),
            pltpu.VMEM((D, n_b), jnp.float32),
            pltpu.VMEM((D, n_b), jnp.float32),
            pltpu.VMEM((n_b,), jnp.int32),
            pltpu.VMEM((n_b,), jnp.int32),
            pltpu.VMEM((n_b,), jnp.int32),
            pltpu.VMEM((n_b,), jnp.int32),
            pltpu.SemaphoreType.DMA,
            pltpu.SemaphoreType.DMA,
        ],
        compiler_params=pltpu.CompilerParams(
            use_tc_tiling_on_sc=True, needs_layout_passes=False),
    )
    def k(w2_hbm, idx_hbm, out_hbm,
          idx_all, rows0, rows1, ot0, ot1, iw0, iw1, po0, po1, sem0, sem1):
        wid = lax.axis_index("s") * NC + lax.axis_index("c")
        pltpu.sync_copy(idx_hbm.at[pl.ds(wid * n_i, n_i)], idx_all)
        rows = (rows0, rows1)
        ots = (ot0, ot1)
        iws = (iw0, iw1)
        pos = (po0, po1)
        sems = (sem0, sem1)

        iota = lax.iota(jnp.int32, L)
        iotaH = iota * H

        def build(h, b):
            for g in range(n_b // L):
                voff = iotaH + (g * L * H + h)
                vo = plsc.load_gather(idx_all, [voff])
                iws[b][pl.ds(g * L, L)] = jnp.right_shift(vo, 1)
                pos[b][pl.ds(g * L, L)] = jnp.bitwise_and(vo, 1) * D

        def fire(b):
            pltpu.async_copy(w2_hbm.at[iws[b]], rows[b], sems[b])

        def compute(h, b):
            rv, ov = rows[b], ots[b]
            ng = n_b // L
            p16s = [pos[b][pl.ds(g * L, L)] for g in range(ng)]

            def dbody(dd, carry):
                cnt = jnp.bitwise_xor(iota, dd)
                for g in range(ng):
                    row16 = iota + g * L
                    r16 = plsc.load_gather(rv, [row16, p16s[g] + cnt])
                    plsc.store_scatter(ov, [cnt, row16], r16)
                return carry

            lax.fori_loop(0, D, dbody, 0, unroll=2)
            pltpu.sync_copy(ov, out_hbm.at[h, :, pl.ds(wid * n_b, n_b)])

        build(0, 0)
        fire(0)

        def half(h, b):
            @pl.when(h + 1 < H)
            def _():
                build(h + 1, 1 - b)
                fire(1 - b)

            pltpu.make_async_copy(
                w2_hbm.at[pl.ds(0, n_b)], rows[b], sems[b]).wait()
            compute(h, b)

        def pair(t, carry):
            half(2 * t, 0)
            half(2 * t + 1, 1)
            return carry

        lax.fori_loop(0, H // 2, pair, 0)

    return k


def kernel(x, W):
    B, H = x.shape
    V, D = W.shape
    keep = jax.random.bernoulli(
        jax.random.key(42), 1.0 - _PROB, (V, 1)
    ).astype(W.dtype) / (1.0 - _PROB)
    keep = keep.reshape(V)
    idx = x.reshape(B * H).astype(jnp.int32)
    ntail = V - V // 128 * 128
    tail = (W[V - ntail:] * keep[V - ntail:, None]).reshape(ntail // 2, 2 * D)
    w2 = _make_format_kernel(V, D)(W.T, keep, tail)
    out3 = _make_gather_kernel(V, D, B, H)(w2, idx)
    return jnp.transpose(out3, (2, 0, 1))


# speedup vs baseline: 1.3847x; 1.2554x over previous
import functools

import jax
import jax.numpy as jnp
import numpy as np
from jax import lax
from jax.experimental import pallas as pl
from jax.experimental.pallas import tpu as pltpu
from jax.experimental.pallas import tpu_sc as plsc

_PROB = 0.1


@functools.cache
def _make_format_kernel(V, D):
    info = plsc.get_sparse_core_info()
    NC, NS, L = info.num_cores, info.num_subcores, info.num_lanes
    NW = NC * NS
    assert D == 64 and L == 16
    NT = V // 128
    ntail = V - NT * 128
    base_n = NT // NW
    extra = NT - base_n * NW
    mesh = plsc.VectorSubcoreMesh(core_axis_name="c", subcore_axis_name="s")

    @functools.partial(
        pl.kernel,
        mesh=mesh,
        out_type=jax.ShapeDtypeStruct((V // 2 + ntail // 2, 2 * D),
                                      jnp.float32),
        scratch_types=[
            pltpu.VMEM((D, 128), jnp.float32),
            pltpu.VMEM((D, 128), jnp.float32),
            pltpu.VMEM((D, 128), jnp.float32),
            pltpu.VMEM((D, 128), jnp.float32),
            pltpu.VMEM((128,), jnp.float32),
            pltpu.VMEM((128,), jnp.float32),
            pltpu.SemaphoreType.DMA,
            pltpu.SemaphoreType.DMA,
            pltpu.SemaphoreType.DMA,
        ],
        compiler_params=pltpu.CompilerParams(
            use_tc_tiling_on_sc=True, needs_layout_passes=False),
    )
    def k(wt_hbm, keep_hbm, tail_hbm, w2_hbm,
          in0, in1, ob0, ob1, kv0, kv1, sem0, sem1, sem_o):
        wid = lax.axis_index("s") * NC + lax.axis_index("c")
        n_j = base_n + jnp.where(wid < extra, 1, 0)
        ins = (in0, in1)
        obs = (ob0, ob1)
        kvs = (kv0, kv1)
        sems = (sem0, sem1)

        iota = lax.iota(jnp.int32, L)

        def vt_of(j):
            return j * NW + wid

        def fire(j, b):
            c = vt_of(j) * 128
            pltpu.async_copy(wt_hbm.at[:, pl.ds(c, 128)], ins[b], sems[b])
            pltpu.async_copy(keep_hbm.at[pl.ds(c, 128)], kvs[b], sems[b])

        def drain_out():
            pltpu.make_async_copy(
                ob0, w2_hbm.at[pl.ds(0, 64)], sem_o).wait()

        def transpose_scale(src, kv, dst, width):
            ng = width // L
            k16s = [kv[pl.ds(g * L, L)] for g in range(ng)]

            def dbody(dd, carry):
                cnt = jnp.bitwise_xor(iota, dd)
                for g in range(ng):
                    row16 = iota + g * L
                    wr16 = jnp.right_shift(row16, 1)
                    par64 = jnp.bitwise_and(row16, 1) * D
                    r16 = plsc.load_gather(src, [cnt, row16])
                    plsc.store_scatter(dst, [wr16, par64 + cnt],
                                       r16 * k16s[g])
                return carry

            lax.fori_loop(0, D, dbody, 0, unroll=2)

        def body(j, b):
            @pl.when(j + 1 < n_j)
            def _():
                fire(j + 1, 1 - b)

            pltpu.make_async_copy(
                wt_hbm.at[:, pl.ds(0, 128)], ins[b], sems[b]).wait()
            pltpu.make_async_copy(
                keep_hbm.at[pl.ds(0, 128)], kvs[b], sems[b]).wait()

            @pl.when(j >= 2)
            def _():
                drain_out()

            transpose_scale(ins[b], kvs[b], obs[b], 128)
            pltpu.async_copy(obs[b], w2_hbm.at[pl.ds(vt_of(j) * 64, 64)],
                             sem_o)

        fire(0, 0)

        def pair(t, carry):
            body(2 * t, 0)
            body(2 * t + 1, 1)
            return carry

        assert base_n % 2 == 0
        lax.fori_loop(0, base_n // 2, pair, 0)
        drain_out()
        drain_out()

        @pl.when(wid < extra)
        def _():
            b = base_n % 2
            pltpu.make_async_copy(
                wt_hbm.at[:, pl.ds(0, 128)], ins[b], sems[b]).wait()
            pltpu.make_async_copy(
                keep_hbm.at[pl.ds(0, 128)], kvs[b], sems[b]).wait()
            transpose_scale(ins[b], kvs[b], obs[b], 128)
            pltpu.sync_copy(obs[b], w2_hbm.at[pl.ds(vt_of(base_n) * 64, 64)])

        @pl.when(wid == 0)
        def _():
            pltpu.sync_copy(tail_hbm, ob0.at[pl.ds(0, ntail // 2)])
            pltpu.sync_copy(ob0.at[pl.ds(0, ntail // 2)],
                            w2_hbm.at[pl.ds(NT * 64, ntail // 2)])

    return k


@functools.cache
def _make_gather_kernel(V, D, B, H):
    info = plsc.get_sparse_core_info()
    NC, NS, L = info.num_cores, info.num_subcores, info.num_lanes
    NW = NC * NS
    assert D == 64 and L == 16 and B % (128 * NW) == 0 and H % 2 == 0
    n_b = B // NW
    n_i = n_b * H
    mesh = plsc.VectorSubcoreMesh(core_axis_name="c", subcore_axis_name="s")

    @functools.partial(
        pl.kernel,
        mesh=mesh,
        out_type=jax.ShapeDtypeStruct((H, D, B), jnp.float32),
        scratch_types=[
            pltpu.VMEM((n_i,), jnp.int32),
            pltpu.VMEM((n_b, 2 * D), jnp.float32),
            pltpu.VMEM((n_b, 2 * D), jnp.float32),
            pltpu.VMEM((D, n_b), jnp.float32),
            pltpu.VMEM((D, n_b), jnp.float32),
            pltpu.VMEM((n_b,), jnp.int32),
            pltpu.VMEM((n_b,), jnp.int32),
            pltpu.VMEM((n_b,), jnp.int32),
            pltpu.VMEM((n_b,), jnp.int32),
            pltpu.SemaphoreType.DMA,
            pltpu.SemaphoreType.DMA,
            pltpu.SemaphoreType.DMA,
        ],
        compiler_params=pltpu.CompilerParams(
            use_tc_tiling_on_sc=True, needs_layout_passes=False),
    )
    def k(w2_hbm, idx_hbm, out_hbm,
          idx_all, rows0, rows1, ot0, ot1, iw0, iw1, po0, po1,
          sem0, sem1, sem_o):
        wid = lax.axis_index("s") * NC + lax.axis_index("c")
        pltpu.sync_copy(idx_hbm.at[pl.ds(wid * n_i, n_i)], idx_all)
        rows = (rows0, rows1)
        ots = (ot0, ot1)
        iws = (iw0, iw1)
        pos = (po0, po1)
        sems = (sem0, sem1)

        iota = lax.iota(jnp.int32, L)
        iotaH = iota * H

        def build(h, b):
            for g in range(n_b // L):
                voff = iotaH + (g * L * H + h)
                vo = plsc.load_gather(idx_all, [voff])
                iws[b][pl.ds(g * L, L)] = jnp.right_shift(vo, 1)
                pos[b][pl.ds(g * L, L)] = jnp.bitwise_and(vo, 1) * D

        def fire(b):
            pltpu.async_copy(w2_hbm.at[iws[b]], rows[b], sems[b])

        def compute(h, b):
            rv, ov = rows[b], ots[b]
            ng = n_b // L
            p16s = [pos[b][pl.ds(g * L, L)] for g in range(ng)]

            def dbody(dd, carry):
                cnt = jnp.bitwise_xor(iota, dd)
                for g in range(ng):
                    row16 = iota + g * L
                    r16 = plsc.load_gather(rv, [row16, p16s[g] + cnt])
                    plsc.store_scatter(ov, [cnt, row16], r16)
                return carry

            @pl.when(h >= 2)
            def _():
                pltpu.make_async_copy(
                    ot0, out_hbm.at[0, :, pl.ds(0, n_b)], sem_o).wait()

            lax.fori_loop(0, D, dbody, 0, unroll=2)
            pltpu.async_copy(ov, out_hbm.at[h, :, pl.ds(wid * n_b, n_b)],
                             sem_o)

        build(0, 0)
        fire(0)

        def half(h, b):
            @pl.when(h + 1 < H)
            def _():
                build(h + 1, 1 - b)
                fire(1 - b)

            pltpu.make_async_copy(
                w2_hbm.at[pl.ds(0, n_b)], rows[b], sems[b]).wait()
            compute(h, b)

        def pair(t, carry):
            half(2 * t, 0)
            half(2 * t + 1, 1)
            return carry

        lax.fori_loop(0, H // 2, pair, 0)
        for _ in range(2):
            pltpu.make_async_copy(
                ot0, out_hbm.at[0, :, pl.ds(0, n_b)], sem_o).wait()

    return k


def kernel(x, W):
    B, H = x.shape
    V, D = W.shape
    keep = jax.random.bernoulli(
        jax.random.key(42), 1.0 - _PROB, (V, 1)
    ).astype(W.dtype) / (1.0 - _PROB)
    keep = keep.reshape(V)
    idx = x.reshape(B * H).astype(jnp.int32)
    ntail = V - V // 128 * 128
    tail = (W[V - ntail:] * keep[V - ntail:, None]).reshape(ntail // 2, 2 * D)
    w2 = _make_format_kernel(V, D)(W.T, keep, tail)
    out3 = _make_gather_kernel(V, D, B, H)(w2, idx)
    return jnp.transpose(out3, (2, 0, 1))
